# Initial kernel scaffold; baseline (speedup 1.0000x reference)
import jax
import jax.numpy as jnp
from jax.experimental import pallas as pl


def kernel(tokens, W_E):
    raise NotImplementedError("write your pallas kernel here")


# speedup vs baseline: 1.4119x; 1.4119x over previous
import functools

import jax
import jax.numpy as jnp
from jax import lax
from jax.experimental import pallas as pl
from jax.experimental.pallas import tpu as pltpu
from jax.experimental.pallas import tpu_sc as plsc

D_VOCAB = 100000
D_MODEL = 768
BATCH = 4
SEQ = 2048

NC = 2
NS = 16
NW = NC * NS

B_TOTAL = BATCH * SEQ
B_PER_W = B_TOTAL // NW
CHUNK = 64
N_CHUNKS = B_PER_W // CHUNK


@functools.partial(
    pl.kernel,
    out_type=jax.ShapeDtypeStruct((B_TOTAL, D_MODEL), jnp.float32),
    mesh=plsc.VectorSubcoreMesh(core_axis_name="c", subcore_axis_name="s"),
    scratch_types=[
        pltpu.VMEM((N_CHUNKS, CHUNK), jnp.int32),
        pltpu.VMEM((CHUNK, D_MODEL), jnp.float32),
        pltpu.SemaphoreType.DMA,
    ],
)
def _embed_sc(idx_hbm, table_hbm, out_hbm, idx_v, rows_v, sem):
    wid = lax.axis_index("s") * NC + lax.axis_index("c")
    base = wid * B_PER_W
    pltpu.sync_copy(idx_hbm.at[wid], idx_v)
    for c in range(N_CHUNKS):
        pltpu.async_copy(table_hbm.at[idx_v.at[c]], rows_v, sem).wait()
        pltpu.sync_copy(rows_v, out_hbm.at[pl.ds(base + c * CHUNK, CHUNK)])


def kernel(tokens, W_E):
    idx = tokens.reshape(NW, N_CHUNKS, CHUNK).astype(jnp.int32)
    out = _embed_sc(idx, W_E)
    return out.reshape(BATCH, SEQ, D_MODEL)


# speedup vs baseline: 1.4733x; 1.0435x over previous
import functools

import jax
import jax.numpy as jnp
from jax import lax
from jax.experimental import pallas as pl
from jax.experimental.pallas import tpu as pltpu
from jax.experimental.pallas import tpu_sc as plsc

D_VOCAB = 100000
D_MODEL = 768
BATCH = 4
SEQ = 2048

NC = 2
NS = 16
NW = NC * NS

B_TOTAL = BATCH * SEQ
B_PER_W = B_TOTAL // NW
CHUNK = 64
N_CHUNKS = B_PER_W // CHUNK


@functools.partial(
    pl.kernel,
    out_type=jax.ShapeDtypeStruct((B_TOTAL, D_MODEL), jnp.float32),
    mesh=plsc.VectorSubcoreMesh(core_axis_name="c", subcore_axis_name="s"),
    scratch_types=[
        pltpu.VMEM((N_CHUNKS, CHUNK), jnp.int32),
        pltpu.VMEM((CHUNK, D_MODEL), jnp.float32),
        pltpu.VMEM((CHUNK, D_MODEL), jnp.float32),
        pltpu.SemaphoreType.DMA,
        pltpu.SemaphoreType.DMA,
        pltpu.SemaphoreType.DMA,
        pltpu.SemaphoreType.DMA,
    ],
)
def _embed_sc(idx_hbm, table_hbm, out_hbm, idx_v, rows0, rows1, sg0, sg1, sw0, sw1):
    wid = lax.axis_index("s") * NC + lax.axis_index("c")
    base = wid * B_PER_W
    pltpu.sync_copy(idx_hbm.at[wid], idx_v)
    bufs = (rows0, rows1)
    sgs = (sg0, sg1)
    sws = (sw0, sw1)
    g = [None] * N_CHUNKS
    w = [None] * N_CHUNKS
    for c in range(min(2, N_CHUNKS)):
        g[c] = pltpu.async_copy(table_hbm.at[idx_v.at[c]], bufs[c % 2], sgs[c % 2])
    for c in range(N_CHUNKS):
        b = c % 2
        g[c].wait()
        w[c] = pltpu.async_copy(
            bufs[b], out_hbm.at[pl.ds(base + c * CHUNK, CHUNK)], sws[b]
        )
        nxt = c + 2
        if nxt < N_CHUNKS:
            w[c].wait()
            g[nxt] = pltpu.async_copy(table_hbm.at[idx_v.at[nxt]], bufs[b], sgs[b])
    for c in range(max(0, N_CHUNKS - 2), N_CHUNKS):
        w[c].wait()


def kernel(tokens, W_E):
    idx = tokens.reshape(NW, N_CHUNKS, CHUNK).astype(jnp.int32)
    out = _embed_sc(idx, W_E)
    return out.reshape(BATCH, SEQ, D_MODEL)


# speedup vs baseline: 1.4832x; 1.0067x over previous
import functools

import jax
import jax.numpy as jnp
from jax import lax
from jax.experimental import pallas as pl
from jax.experimental.pallas import tpu as pltpu
from jax.experimental.pallas import tpu_sc as plsc

D_VOCAB = 100000
D_MODEL = 768
BATCH = 4
SEQ = 2048

NC = 2
NS = 16
NW = NC * NS

B_TOTAL = BATCH * SEQ
B_PER_W = B_TOTAL // NW
W_PER_BATCH = NW // BATCH
CHUNK = 64
N_CHUNKS = B_PER_W // CHUNK


@functools.partial(
    pl.kernel,
    out_type=jax.ShapeDtypeStruct((BATCH, SEQ, D_MODEL), jnp.float32),
    mesh=plsc.VectorSubcoreMesh(core_axis_name="c", subcore_axis_name="s"),
    scratch_types=[
        pltpu.VMEM((B_PER_W,), jnp.int32),
        pltpu.VMEM((CHUNK, D_MODEL), jnp.float32),
        pltpu.VMEM((CHUNK, D_MODEL), jnp.float32),
        pltpu.SemaphoreType.DMA,
        pltpu.SemaphoreType.DMA,
        pltpu.SemaphoreType.DMA,
        pltpu.SemaphoreType.DMA,
    ],
)
def _embed_sc(idx_hbm, table_hbm, out_hbm, idx_v, rows0, rows1, sg0, sg1, sw0, sw1):
    wid = lax.axis_index("s") * NC + lax.axis_index("c")
    bi = wid // W_PER_BATCH
    s0 = (wid % W_PER_BATCH) * B_PER_W
    pltpu.sync_copy(idx_hbm.at[bi, pl.ds(s0, B_PER_W)], idx_v)
    bufs = (rows0, rows1)
    sgs = (sg0, sg1)
    sws = (sw0, sw1)
    g = [None] * N_CHUNKS
    w = [None] * N_CHUNKS
    for c in range(min(2, N_CHUNKS)):
        g[c] = pltpu.async_copy(
            table_hbm.at[idx_v.at[pl.ds(c * CHUNK, CHUNK)]], bufs[c % 2], sgs[c % 2]
        )
    for c in range(N_CHUNKS):
        b = c % 2
        g[c].wait()
        w[c] = pltpu.async_copy(
            bufs[b], out_hbm.at[bi].at[pl.ds(s0 + c * CHUNK, CHUNK)], sws[b]
        )
        nxt = c + 2
        if nxt < N_CHUNKS:
            w[c].wait()
            g[nxt] = pltpu.async_copy(
                table_hbm.at[idx_v.at[pl.ds(nxt * CHUNK, CHUNK)]], bufs[b], sgs[b]
            )
    for c in range(max(0, N_CHUNKS - 2), N_CHUNKS):
        w[c].wait()


def kernel(tokens, W_E):
    return _embed_sc(tokens, W_E)


# speedup vs baseline: 1.5166x; 1.0225x over previous
import functools

import jax
import jax.numpy as jnp
from jax import lax
from jax.experimental import pallas as pl
from jax.experimental.pallas import tpu as pltpu
from jax.experimental.pallas import tpu_sc as plsc

D_VOCAB = 100000
D_MODEL = 768
BATCH = 4
SEQ = 2048

NC = 2
NS = 16
NW = NC * NS

B_TOTAL = BATCH * SEQ
B_PER_W = B_TOTAL // NW
W_PER_BATCH = NW // BATCH
CHUNK = 32
N_CHUNKS = B_PER_W // CHUNK
NBUF = 4
LEAD = NBUF - 1


@functools.partial(
    pl.kernel,
    out_type=jax.ShapeDtypeStruct((BATCH, SEQ, D_MODEL), jnp.float32),
    mesh=plsc.VectorSubcoreMesh(core_axis_name="c", subcore_axis_name="s"),
    scratch_types=(
        [pltpu.VMEM((B_PER_W,), jnp.int32)]
        + [pltpu.VMEM((CHUNK, D_MODEL), jnp.float32) for _ in range(NBUF)]
        + [pltpu.SemaphoreType.DMA for _ in range(2 * NBUF)]
    ),
)
def _embed_sc(idx_hbm, table_hbm, out_hbm, idx_v, *bufs_and_sems):
    bufs = bufs_and_sems[:NBUF]
    sgs = bufs_and_sems[NBUF : 2 * NBUF]
    sws = bufs_and_sems[2 * NBUF :]
    wid = lax.axis_index("s") * NC + lax.axis_index("c")
    bi = wid // W_PER_BATCH
    s0 = (wid % W_PER_BATCH) * B_PER_W
    pltpu.sync_copy(idx_hbm.at[bi, pl.ds(s0, B_PER_W)], idx_v)

    def gather(c):
        b = c % NBUF
        return pltpu.async_copy(
            table_hbm.at[idx_v.at[pl.ds(c * CHUNK, CHUNK)]], bufs[b], sgs[b]
        )

    g = [None] * N_CHUNKS
    w = [None] * N_CHUNKS
    waited = set()
    for c in range(min(LEAD, N_CHUNKS)):
        g[c] = gather(c)
    for c in range(N_CHUNKS):
        b = c % NBUF
        g[c].wait()
        w[c] = pltpu.async_copy(
            bufs[b], out_hbm.at[bi].at[pl.ds(s0 + c * CHUNK, CHUNK)], sws[b]
        )
        nxt = c + LEAD
        if nxt < N_CHUNKS:
            prev = nxt - NBUF
            if prev >= 0:
                w[prev].wait()
                waited.add(prev)
            g[nxt] = gather(nxt)
    for c in range(N_CHUNKS):
        if c not in waited:
            w[c].wait()


def kernel(tokens, W_E):
    return _embed_sc(tokens, W_E)


# speedup vs baseline: 1.5183x; 1.0011x over previous
import functools

import jax
import jax.numpy as jnp
from jax import lax
from jax.experimental import pallas as pl
from jax.experimental.pallas import tpu as pltpu
from jax.experimental.pallas import tpu_sc as plsc

D_VOCAB = 100000
D_MODEL = 768
BATCH = 4
SEQ = 2048

NC = 2
NS = 16
NW = NC * NS

B_TOTAL = BATCH * SEQ
B_PER_W = B_TOTAL // NW
W_PER_BATCH = NW // BATCH
CHUNK = 16
N_CHUNKS = B_PER_W // CHUNK
NBUF = 8
LEAD = NBUF - 1


@functools.partial(
    pl.kernel,
    out_type=jax.ShapeDtypeStruct((BATCH, SEQ, D_MODEL), jnp.float32),
    mesh=plsc.VectorSubcoreMesh(core_axis_name="c", subcore_axis_name="s"),
    scratch_types=(
        [pltpu.VMEM((B_PER_W,), jnp.int32)]
        + [pltpu.VMEM((CHUNK, D_MODEL), jnp.float32) for _ in range(NBUF)]
        + [pltpu.SemaphoreType.DMA for _ in range(2 * NBUF)]
    ),
)
def _embed_sc(idx_hbm, table_hbm, out_hbm, idx_v, *bufs_and_sems):
    bufs = bufs_and_sems[:NBUF]
    sgs = bufs_and_sems[NBUF : 2 * NBUF]
    sws = bufs_and_sems[2 * NBUF :]
    wid = lax.axis_index("s") * NC + lax.axis_index("c")
    bi = wid // W_PER_BATCH
    s0 = (wid % W_PER_BATCH) * B_PER_W
    pltpu.sync_copy(idx_hbm.at[bi, pl.ds(s0, B_PER_W)], idx_v)

    def gather(c):
        b = c % NBUF
        return pltpu.async_copy(
            table_hbm.at[idx_v.at[pl.ds(c * CHUNK, CHUNK)]], bufs[b], sgs[b]
        )

    g = [None] * N_CHUNKS
    w = [None] * N_CHUNKS
    waited = set()
    for c in range(min(LEAD, N_CHUNKS)):
        g[c] = gather(c)
    for c in range(N_CHUNKS):
        b = c % NBUF
        g[c].wait()
        w[c] = pltpu.async_copy(
            bufs[b], out_hbm.at[bi].at[pl.ds(s0 + c * CHUNK, CHUNK)], sws[b]
        )
        nxt = c + LEAD
        if nxt < N_CHUNKS:
            prev = nxt - NBUF
            if prev >= 0:
                w[prev].wait()
                waited.add(prev)
            g[nxt] = gather(nxt)
    for c in range(N_CHUNKS):
        if c not in waited:
            w[c].wait()


def kernel(tokens, W_E):
    return _embed_sc(tokens, W_E)


# speedup vs baseline: 1.5237x; 1.0035x over previous
import functools

import jax
import jax.numpy as jnp
from jax import lax
from jax.experimental import pallas as pl
from jax.experimental.pallas import tpu as pltpu
from jax.experimental.pallas import tpu_sc as plsc

D_VOCAB = 100000
D_MODEL = 768
BATCH = 4
SEQ = 2048

NC = 2
NS = 16
NW = NC * NS

B_TOTAL = BATCH * SEQ
B_PER_W = B_TOTAL // NW
W_PER_BATCH = NW // BATCH
CHUNK = 32
N_CHUNKS = B_PER_W // CHUNK
NBUF = 5
LEAD = NBUF - 1


@functools.partial(
    pl.kernel,
    out_type=jax.ShapeDtypeStruct((BATCH, SEQ, D_MODEL), jnp.float32),
    mesh=plsc.VectorSubcoreMesh(core_axis_name="c", subcore_axis_name="s"),
    scratch_types=(
        [pltpu.VMEM((B_PER_W,), jnp.int32)]
        + [pltpu.VMEM((CHUNK, D_MODEL), jnp.float32) for _ in range(NBUF)]
        + [pltpu.SemaphoreType.DMA for _ in range(2 * NBUF)]
    ),
)
def _embed_sc(idx_hbm, table_hbm, out_hbm, idx_v, *bufs_and_sems):
    bufs = bufs_and_sems[:NBUF]
    sgs = bufs_and_sems[NBUF : 2 * NBUF]
    sws = bufs_and_sems[2 * NBUF :]
    wid = lax.axis_index("s") * NC + lax.axis_index("c")
    bi = wid // W_PER_BATCH
    s0 = (wid % W_PER_BATCH) * B_PER_W
    pltpu.sync_copy(idx_hbm.at[bi, pl.ds(s0, B_PER_W)], idx_v)

    def gather(c):
        b = c % NBUF
        return pltpu.async_copy(
            table_hbm.at[idx_v.at[pl.ds(c * CHUNK, CHUNK)]], bufs[b], sgs[b]
        )

    g = [None] * N_CHUNKS
    w = [None] * N_CHUNKS
    waited = set()
    for c in range(min(LEAD, N_CHUNKS)):
        g[c] = gather(c)
    for c in range(N_CHUNKS):
        b = c % NBUF
        g[c].wait()
        w[c] = pltpu.async_copy(
            bufs[b], out_hbm.at[bi].at[pl.ds(s0 + c * CHUNK, CHUNK)], sws[b]
        )
        nxt = c + LEAD
        if nxt < N_CHUNKS:
            prev = nxt - NBUF
            if prev >= 0:
                w[prev].wait()
                waited.add(prev)
            g[nxt] = gather(nxt)
    for c in range(N_CHUNKS):
        if c not in waited:
            w[c].wait()


def kernel(tokens, W_E):
    return _embed_sc(tokens, W_E)
